# Initial kernel scaffold; baseline (speedup 1.0000x reference)
import jax
import jax.numpy as jnp
from jax.experimental import pallas as pl


def kernel(x_user, x_item):
    raise NotImplementedError("write your pallas kernel here")


# speedup vs baseline: 1.9598x; 1.9598x over previous
import jax
import jax.numpy as jnp
from jax.experimental import pallas as pl


_B = 8


def _cross_body(xu_ref, xi_ref, o_ref):
    xu = xu_ref[...]
    xi = xi_ref[...]
    b, u, e = xu.shape
    i = xi.shape[1]
    ou = jnp.broadcast_to(xu[:, :, None, :], (b, u, i, e)).reshape(b, u * i, e)
    oi = jnp.broadcast_to(xi[:, None, :, :], (b, u, i, e)).reshape(b, u * i, e)
    o_ref[...] = jnp.concatenate([ou, oi], axis=-1)


def kernel(x_user, x_item):
    n, u, e = x_user.shape
    i = x_item.shape[1]
    grid = (n // _B,)
    return pl.pallas_call(
        _cross_body,
        grid=grid,
        in_specs=[
            pl.BlockSpec((_B, u, e), lambda g: (g, 0, 0)),
            pl.BlockSpec((_B, i, e), lambda g: (g, 0, 0)),
        ],
        out_specs=pl.BlockSpec((_B, u * i, 2 * e), lambda g: (g, 0, 0)),
        out_shape=jax.ShapeDtypeStruct((n, u * i, 2 * e), jnp.float32),
    )(x_user, x_item)
